# baseline (speedup 1.0000x reference)
import functools
import math

import jax
import jax.numpy as jnp
from jax import lax
from jax.experimental import pallas as pl
from jax.experimental.pallas import tpu as pltpu
from jax.experimental.pallas import tpu_sc as plsc

HIDDEN = 1024
LANES = 16
VECS_PER_ROW = HIDDEN // LANES
SCALE = math.sqrt(HIDDEN)


def _make_sc_kernel(rows, chunk):
    info = plsc.get_sparse_core_info()
    nc, ns = info.num_cores, info.num_subcores
    nw = nc * ns
    rpw = rows // nw
    n_chunks = rpw // chunk
    mesh = plsc.VectorSubcoreMesh(core_axis_name="c", subcore_axis_name="s")

    @functools.partial(
        pl.kernel,
        mesh=mesh,
        out_type=jax.ShapeDtypeStruct((rows, HIDDEN), jnp.float32),
        scratch_types=[
            pltpu.VMEM((rpw,), jnp.int32),
            pltpu.VMEM((chunk, HIDDEN), jnp.float32),
            pltpu.VMEM((chunk, HIDDEN), jnp.float32),
            pltpu.SemaphoreType.DMA,
            pltpu.SemaphoreType.DMA,
        ],
    )
    def pe_add(x_hbm, idx_hbm, pe_hbm, out_hbm, idx_v, xbuf, pebuf, gsem, osem):
        wid = lax.axis_index("s") * nc + lax.axis_index("c")
        base = wid * rpw
        pltpu.sync_copy(idx_hbm.at[pl.ds(base, rpw)], idx_v)

        def chunk_body(i, _):
            row0 = base + i * chunk
            gcopy = pltpu.async_copy(
                pe_hbm.at[idx_v.at[pl.ds(i * chunk, chunk)]], pebuf, gsem
            )
            pltpu.sync_copy(x_hbm.at[pl.ds(row0, chunk)], xbuf)
            gcopy.wait()

            def row_body(r, _):
                def col_body(c, _):
                    sl = pl.ds(c * LANES, LANES)
                    xbuf[r, sl] = xbuf[r, sl] * SCALE + pebuf[r, sl]
                    return 0

                lax.fori_loop(0, VECS_PER_ROW, col_body, 0)
                return 0

            lax.fori_loop(0, chunk, row_body, 0)
            pltpu.async_copy(xbuf, out_hbm.at[pl.ds(row0, chunk)], osem).wait()
            return 0

        lax.fori_loop(0, n_chunks, chunk_body, 0)

    return pe_add


def kernel(x, indices, pe):
    b, t, d = x.shape
    rows = b * t
    x2 = x.reshape(rows, d)
    idx = jnp.asarray(indices, jnp.int32).reshape(rows)
    out = _make_sc_kernel(rows, 16)(x2, idx, pe)
    return out.reshape(b, t, d)


# speedup vs baseline: 1.1550x; 1.1550x over previous
import functools
import math

import jax
import jax.numpy as jnp
from jax import lax
from jax.experimental import pallas as pl
from jax.experimental.pallas import tpu as pltpu
from jax.experimental.pallas import tpu_sc as plsc

HIDDEN = 1024
LANES = 16
VECS_PER_ROW = HIDDEN // LANES
SCALE = math.sqrt(HIDDEN)


def _make_sc_kernel(rows, chunk):
    info = plsc.get_sparse_core_info()
    nc, ns = info.num_cores, info.num_subcores
    nw = nc * ns
    rpw = rows // nw
    n_chunks = rpw // chunk
    mesh = plsc.VectorSubcoreMesh(core_axis_name="c", subcore_axis_name="s")

    @functools.partial(
        pl.kernel,
        mesh=mesh,
        out_type=jax.ShapeDtypeStruct((rows, HIDDEN), jnp.float32),
        scratch_types=[
            pltpu.VMEM((rpw,), jnp.int32),
            pltpu.VMEM((chunk, HIDDEN), jnp.float32),
            pltpu.VMEM((chunk, HIDDEN), jnp.float32),
            pltpu.SemaphoreType.DMA,
            pltpu.SemaphoreType.DMA,
        ],
    )
    def pe_add(x_hbm, idx_hbm, pe_hbm, out_hbm, idx_v, xbuf, pebuf, gsem, osem):
        wid = lax.axis_index("s") * nc + lax.axis_index("c")
        base = wid * rpw
        pltpu.sync_copy(idx_hbm.at[pl.ds(base, rpw)], idx_v)

        def chunk_body(i, _):
            row0 = base + i * chunk
            gcopy = pltpu.async_copy(
                pe_hbm.at[idx_v.at[pl.ds(i * chunk, chunk)]], pebuf, gsem
            )
            pltpu.sync_copy(x_hbm.at[pl.ds(row0, chunk)], xbuf)
            gcopy.wait()

            def row_body(r, _):
                for c in range(VECS_PER_ROW):
                    sl = pl.ds(c * LANES, LANES)
                    plsc.addupdate(pebuf.at[r, sl], xbuf[r, sl] * SCALE)
                return 0

            lax.fori_loop(0, chunk, row_body, 0)
            pltpu.async_copy(pebuf, out_hbm.at[pl.ds(row0, chunk)], osem).wait()
            return 0

        lax.fori_loop(0, n_chunks, chunk_body, 0)

    return pe_add


def kernel(x, indices, pe):
    b, t, d = x.shape
    rows = b * t
    x2 = x.reshape(rows, d)
    idx = jnp.asarray(indices, jnp.int32).reshape(rows)
    out = _make_sc_kernel(rows, 16)(x2, idx, pe)
    return out.reshape(b, t, d)
